# Initial kernel scaffold; baseline (speedup 1.0000x reference)
import jax
import jax.numpy as jnp
from jax.experimental import pallas as pl


def kernel(categories, table, W, b):
    raise NotImplementedError("write your pallas kernel here")


# speedup vs baseline: 4.8500x; 4.8500x over previous
import functools

import jax
import jax.numpy as jnp
from jax import lax
from jax.experimental import pallas as pl
from jax.experimental.pallas import tpu as pltpu
from jax.experimental.pallas import tpu_sc as plsc

_CHUNK = 128


def _project_table(table, Wt, bias):
    V, E = table.shape
    O = Wt.shape[1]
    VB = 5000 if V % 5000 == 0 else V
    grid = V // VB

    def body(t_ref, w_ref, b_ref, o_ref):
        o_ref[...] = (
            jnp.dot(t_ref[...], w_ref[...], preferred_element_type=jnp.float32)
            + b_ref[...]
        )

    return pl.pallas_call(
        body,
        grid=(grid,),
        in_specs=[
            pl.BlockSpec((VB, E), lambda i: (i, 0)),
            pl.BlockSpec((E, O), lambda i: (0, 0)),
            pl.BlockSpec((1, O), lambda i: (0, 0)),
        ],
        out_specs=pl.BlockSpec((VB, O), lambda i: (i, 0)),
        out_shape=jax.ShapeDtypeStruct((V, O), jnp.float32),
    )(table, Wt, bias.reshape(1, O))


def _gather_rows(proj, idx_flat):
    V, O = proj.shape
    N = idx_flat.shape[0]
    info = plsc.get_sparse_core_info()
    NC, NS = info.num_cores, info.num_subcores
    NW = NC * NS
    n_per_w = N // NW
    n_chunks = n_per_w // _CHUNK
    assert n_per_w * NW == N and n_chunks * _CHUNK == n_per_w

    idx3 = idx_flat.reshape(NW, n_chunks, _CHUNK)
    mesh = plsc.VectorSubcoreMesh(core_axis_name="c", subcore_axis_name="s")

    @functools.partial(
        pl.kernel,
        out_type=jax.ShapeDtypeStruct((N, O), jnp.float32),
        mesh=mesh,
        scratch_types=[
            pltpu.VMEM((n_chunks, _CHUNK), jnp.int32),
            pltpu.VMEM((_CHUNK, O), jnp.float32),
            pltpu.SemaphoreType.DMA,
        ],
        compiler_params=pltpu.CompilerParams(use_tc_tiling_on_sc=False),
    )
    def gk(proj_hbm, idx_hbm, out_hbm, idx_v, buf, gsem):
        wid = lax.axis_index("s") * NC + lax.axis_index("c")
        base = wid * n_per_w
        pltpu.sync_copy(idx_hbm.at[wid], idx_v)

        def body(j, carry):
            pltpu.async_copy(proj_hbm.at[idx_v.at[j]], buf, gsem).wait()
            pltpu.sync_copy(buf, out_hbm.at[pl.ds(base + j * _CHUNK, _CHUNK)])
            return carry

        lax.fori_loop(0, n_chunks, body, 0)

    return gk(proj, idx3)


def kernel(categories, table, W, b):
    B, L = categories.shape
    O = W.shape[0]
    proj = _project_table(table, W.T, b)
    idx = categories.reshape(-1).astype(jnp.int32)
    out = _gather_rows(proj, idx)
    return out.reshape(B, L, O)


# speedup vs baseline: 5.3988x; 1.1131x over previous
import functools

import jax
import jax.numpy as jnp
from jax import lax
from jax.experimental import pallas as pl
from jax.experimental.pallas import tpu as pltpu
from jax.experimental.pallas import tpu_sc as plsc

_NBUF = 4


def _project_table(table, Wt, bias):
    V, E = table.shape
    O = Wt.shape[1]
    VB = 5000 if V % 5000 == 0 else V
    grid = V // VB

    def body(t_ref, w_ref, b_ref, o_ref):
        o_ref[...] = (
            jnp.dot(t_ref[...], w_ref[...], preferred_element_type=jnp.float32)
            + b_ref[...]
        )

    return pl.pallas_call(
        body,
        grid=(grid,),
        in_specs=[
            pl.BlockSpec((VB, E), lambda i: (i, 0)),
            pl.BlockSpec((E, O), lambda i: (0, 0)),
            pl.BlockSpec((1, O), lambda i: (0, 0)),
        ],
        out_specs=pl.BlockSpec((VB, O), lambda i: (i, 0)),
        out_shape=jax.ShapeDtypeStruct((V, O), jnp.float32),
    )(table, Wt, bias.reshape(1, O))


def _gather_rows(proj, idx3, B, L):
    V, O = proj.shape
    NW, b_per_w, _ = idx3.shape
    mesh = plsc.VectorSubcoreMesh(core_axis_name="c", subcore_axis_name="s")
    info = plsc.get_sparse_core_info()
    NC = info.num_cores

    @functools.partial(
        pl.kernel,
        out_type=jax.ShapeDtypeStruct((B, L, O), jnp.float32),
        mesh=mesh,
        scratch_types=[
            pltpu.VMEM((b_per_w, L), jnp.int32),
            pltpu.VMEM((_NBUF, L, O), jnp.float32),
            [pltpu.SemaphoreType.DMA] * _NBUF,
            [pltpu.SemaphoreType.DMA] * _NBUF,
        ],
        compiler_params=pltpu.CompilerParams(use_tc_tiling_on_sc=False),
    )
    def gk(proj_hbm, idx_hbm, out_hbm, idx_v, buf, gsems, osems):
        wid = lax.axis_index("s") * NC + lax.axis_index("c")
        base = wid * b_per_w
        pltpu.sync_copy(idx_hbm.at[wid], idx_v)

        def start_gather(j, s):
            pltpu.async_copy(proj_hbm.at[idx_v.at[j]], buf.at[s], gsems[s])

        def wait_gather(j, s):
            pltpu.make_async_copy(proj_hbm.at[idx_v.at[j]], buf.at[s],
                                  gsems[s]).wait()

        def start_out(j, s):
            pltpu.async_copy(buf.at[s], out_hbm.at[base + j], osems[s])

        def wait_out(j, s):
            pltpu.make_async_copy(buf.at[s], out_hbm.at[base + j],
                                  osems[s]).wait()

        for s in range(_NBUF):
            start_gather(s, s)

        def body(i, carry):
            for s in range(_NBUF):
                j = i * _NBUF + s
                wait_gather(j, s)
                start_out(j, s)
            for s in range(_NBUF):
                j = i * _NBUF + s
                wait_out(j, s)
                start_gather(j + _NBUF, s)
            return carry

        n_grp = b_per_w // _NBUF
        lax.fori_loop(0, n_grp - 1, body, 0)
        for s in range(_NBUF):
            j = (n_grp - 1) * _NBUF + s
            wait_gather(j, s)
            start_out(j, s)
        for s in range(_NBUF):
            j = (n_grp - 1) * _NBUF + s
            wait_out(j, s)

    return gk(proj, idx3)


def kernel(categories, table, W, b):
    B, L = categories.shape
    O = W.shape[0]
    info = plsc.get_sparse_core_info()
    NW = info.num_cores * info.num_subcores
    proj = _project_table(table, W.T, b)
    idx3 = categories.astype(jnp.int32).reshape(NW, B // NW, L)
    return _gather_rows(proj, idx3, B, L)


# speedup vs baseline: 6.3321x; 1.1729x over previous
import functools

import jax
import jax.numpy as jnp
from jax import lax
from jax.experimental import pallas as pl
from jax.experimental.pallas import tpu as pltpu
from jax.experimental.pallas import tpu_sc as plsc

_NBUF = 4
_LANE = 128


def _project_table(table, Wt_pad, bias_pad):
    V, E = table.shape
    P = Wt_pad.shape[1]
    VB = 5000 if V % 5000 == 0 else V
    grid = V // VB

    def body(t_ref, w_ref, b_ref, o_ref):
        o_ref[...] = (
            jnp.dot(t_ref[...], w_ref[...], preferred_element_type=jnp.float32)
            + b_ref[...]
        )

    return pl.pallas_call(
        body,
        grid=(grid,),
        in_specs=[
            pl.BlockSpec((VB, E), lambda i: (i, 0)),
            pl.BlockSpec((E, P), lambda i: (0, 0)),
            pl.BlockSpec((1, P), lambda i: (0, 0)),
        ],
        out_specs=pl.BlockSpec((VB, P), lambda i: (i, 0)),
        out_shape=jax.ShapeDtypeStruct((V, P), jnp.float32),
    )(table, Wt_pad, bias_pad.reshape(1, P))


def _gather_rows(proj, cats, O):
    B, L = cats.shape
    mesh = plsc.VectorSubcoreMesh(core_axis_name="c", subcore_axis_name="s")
    info = plsc.get_sparse_core_info()
    NC = info.num_cores
    NW = NC * info.num_subcores
    b_per_w = B // NW

    @functools.partial(
        pl.kernel,
        out_type=jax.ShapeDtypeStruct((B, L, O), jnp.float32),
        mesh=mesh,
        scratch_types=[
            pltpu.VMEM((b_per_w, L), jnp.int32),
            pltpu.VMEM((_NBUF, 56, _LANE), jnp.float32),
            pltpu.VMEM((_NBUF, 50, 64), jnp.float32),
            [pltpu.SemaphoreType.DMA] * _NBUF,
            [pltpu.SemaphoreType.DMA] * _NBUF,
        ],
        compiler_params=pltpu.CompilerParams(use_tc_tiling_on_sc=True),
    )
    def gk(proj_hbm, cats_hbm, out_hbm, idx_v, buf, bufb, gsems, osems):
        wid = lax.axis_index("s") * NC + lax.axis_index("c")
        base = wid * b_per_w
        pltpu.sync_copy(cats_hbm.at[pl.ds(base, b_per_w)], idx_v)

        def start_gather(j, s):
            pltpu.async_copy(proj_hbm.at[idx_v.at[j]], buf.at[s, :L], gsems[s])

        def wait_gather(j, s):
            pltpu.make_async_copy(proj_hbm.at[idx_v.at[j]], buf.at[s, :L],
                                  gsems[s]).wait()

        def repack(s):
            def rbody(r, carry):
                for c in range(O // 16):
                    bufb[s, r, pl.ds(c * 16, 16)] = buf[s, r, pl.ds(c * 16, 16)]
                return carry

            lax.fori_loop(0, L, rbody, 0)

        def start_out(j, s):
            pltpu.async_copy(bufb.at[s], out_hbm.at[base + j], osems[s])

        def wait_out(j, s):
            pltpu.make_async_copy(bufb.at[s], out_hbm.at[base + j],
                                  osems[s]).wait()

        for s in range(_NBUF):
            start_gather(s, s)

        def body(i, carry):
            for s in range(_NBUF):
                j = i * _NBUF + s
                wait_gather(j, s)
                repack(s)
                start_out(j, s)
            for s in range(_NBUF):
                j = i * _NBUF + s
                wait_out(j, s)
                start_gather(j + _NBUF, s)
            return carry

        n_grp = b_per_w // _NBUF
        lax.fori_loop(0, n_grp - 1, body, 0)
        for s in range(_NBUF):
            j = (n_grp - 1) * _NBUF + s
            wait_gather(j, s)
            repack(s)
            start_out(j, s)
        for s in range(_NBUF):
            j = (n_grp - 1) * _NBUF + s
            wait_out(j, s)

    return gk(proj, cats)


def kernel(categories, table, W, b):
    O, E = W.shape
    Wt_pad = jnp.pad(W.T, ((0, 0), (0, _LANE - O)))
    bias_pad = jnp.pad(b, (0, _LANE - O))
    proj = _project_table(table, Wt_pad, bias_pad)
    cats = categories.astype(jnp.int32)
    return _gather_rows(proj, cats, O)


# speedup vs baseline: 7.1008x; 1.1214x over previous
import functools

import jax
import jax.numpy as jnp
from jax import lax
from jax.experimental import pallas as pl
from jax.experimental.pallas import tpu as pltpu
from jax.experimental.pallas import tpu_sc as plsc

_NBUF = 4
_LANE = 128
_BB = 512


def _gather_rows(tpad, cats, E):
    B, L = cats.shape
    mesh = plsc.VectorSubcoreMesh(core_axis_name="c", subcore_axis_name="s")
    info = plsc.get_sparse_core_info()
    NC = info.num_cores
    NW = NC * info.num_subcores
    b_per_w = B // NW

    @functools.partial(
        pl.kernel,
        out_type=jax.ShapeDtypeStruct((B, L, E), jnp.float32),
        mesh=mesh,
        scratch_types=[
            pltpu.VMEM((b_per_w, L), jnp.int32),
            pltpu.VMEM((_NBUF, 56, _LANE), jnp.float32),
            pltpu.VMEM((_NBUF, L, E), jnp.float32),
            [pltpu.SemaphoreType.DMA] * _NBUF,
            [pltpu.SemaphoreType.DMA] * _NBUF,
        ],
        compiler_params=pltpu.CompilerParams(use_tc_tiling_on_sc=True),
    )
    def gk(tpad_hbm, cats_hbm, out_hbm, idx_v, buf, bufb, gsems, osems):
        wid = lax.axis_index("s") * NC + lax.axis_index("c")
        base = wid * b_per_w
        pltpu.sync_copy(cats_hbm.at[pl.ds(base, b_per_w)], idx_v)

        def start_gather(j, s):
            pltpu.async_copy(tpad_hbm.at[idx_v.at[j]], buf.at[s, :L], gsems[s])

        def wait_gather(j, s):
            pltpu.make_async_copy(tpad_hbm.at[idx_v.at[j]], buf.at[s, :L],
                                  gsems[s]).wait()

        def repack(s):
            def rbody(r, carry):
                for c in range(E // 16):
                    bufb[s, r, pl.ds(c * 16, 16)] = buf[s, r, pl.ds(c * 16, 16)]
                return carry

            lax.fori_loop(0, L, rbody, 0)

        def start_out(j, s):
            pltpu.async_copy(bufb.at[s], out_hbm.at[base + j], osems[s])

        def wait_out(j, s):
            pltpu.make_async_copy(bufb.at[s], out_hbm.at[base + j],
                                  osems[s]).wait()

        for s in range(_NBUF):
            start_gather(s, s)

        def body(i, carry):
            for s in range(_NBUF):
                j = i * _NBUF + s
                wait_gather(j, s)
                repack(s)
                start_out(j, s)
            for s in range(_NBUF):
                j = i * _NBUF + s
                wait_out(j, s)
                start_gather(j + _NBUF, s)
            return carry

        n_grp = b_per_w // _NBUF
        lax.fori_loop(0, n_grp - 1, body, 0)
        for s in range(_NBUF):
            j = (n_grp - 1) * _NBUF + s
            wait_gather(j, s)
            repack(s)
            start_out(j, s)
        for s in range(_NBUF):
            j = (n_grp - 1) * _NBUF + s
            wait_out(j, s)

    return gk(tpad, cats)


def _project_transpose(g, Wt, bias_col):
    B, L, E = g.shape
    O = Wt.shape[1]

    def body(g_ref, w_ref, b_ref, o_ref):
        w = w_ref[...]
        bb = b_ref[...]
        for l in range(L):
            x = g_ref[:, l, :]
            y = lax.dot_general(w, x, (((0,), (1,)), ((), ())),
                                preferred_element_type=jnp.float32)
            o_ref[l] = y + bb

    return pl.pallas_call(
        body,
        grid=(B // _BB,),
        in_specs=[
            pl.BlockSpec((_BB, L, E), lambda i: (i, 0, 0)),
            pl.BlockSpec((E, O), lambda i: (0, 0)),
            pl.BlockSpec((O, 1), lambda i: (0, 0)),
        ],
        out_specs=pl.BlockSpec((L, O, _BB), lambda i: (0, 0, i)),
        out_shape=jax.ShapeDtypeStruct((L, O, B), jnp.float32),
    )(g, Wt, bias_col)


def kernel(categories, table, W, b):
    O, E = W.shape
    tpad = jnp.pad(table, ((0, 0), (0, _LANE - E)))
    cats = categories.astype(jnp.int32)
    g = _gather_rows(tpad, cats, E)
    x = _project_transpose(g, W.T, b.reshape(O, 1))
    return jnp.transpose(x, (2, 0, 1))


# speedup vs baseline: 7.4132x; 1.0440x over previous
import functools

import jax
import jax.numpy as jnp
from jax import lax
from jax.experimental import pallas as pl
from jax.experimental.pallas import tpu as pltpu
from jax.experimental.pallas import tpu_sc as plsc

_NBUF = 4
_LANE = 128
_BB = 512
_NCHUNK = 2


def _gather_rows(tpad, cats, E, b0, nb):
    B, L = cats.shape
    mesh = plsc.VectorSubcoreMesh(core_axis_name="c", subcore_axis_name="s")
    info = plsc.get_sparse_core_info()
    NC = info.num_cores
    NW = NC * info.num_subcores
    b_per_w = nb // NW

    @functools.partial(
        pl.kernel,
        out_type=jax.ShapeDtypeStruct((nb, L, E), jnp.float32),
        mesh=mesh,
        scratch_types=[
            pltpu.VMEM((b_per_w, L), jnp.int32),
            pltpu.VMEM((_NBUF, 56, _LANE), jnp.float32),
            pltpu.VMEM((_NBUF, L, E), jnp.float32),
            [pltpu.SemaphoreType.DMA] * _NBUF,
            [pltpu.SemaphoreType.DMA] * _NBUF,
        ],
        compiler_params=pltpu.CompilerParams(use_tc_tiling_on_sc=True),
    )
    def gk(tpad_hbm, cats_hbm, out_hbm, idx_v, buf, bufb, gsems, osems):
        wid = lax.axis_index("s") * NC + lax.axis_index("c")
        base = wid * b_per_w
        pltpu.sync_copy(cats_hbm.at[pl.ds(b0 + base, b_per_w)], idx_v)

        def start_gather(j, s):
            pltpu.async_copy(tpad_hbm.at[idx_v.at[j]], buf.at[s, :L], gsems[s])

        def wait_gather(j, s):
            pltpu.make_async_copy(tpad_hbm.at[idx_v.at[j]], buf.at[s, :L],
                                  gsems[s]).wait()

        def repack(s):
            def rbody(r, carry):
                for c in range(E // 16):
                    bufb[s, r, pl.ds(c * 16, 16)] = buf[s, r, pl.ds(c * 16, 16)]
                return carry

            lax.fori_loop(0, L, rbody, 0)

        def start_out(j, s):
            pltpu.async_copy(bufb.at[s], out_hbm.at[base + j], osems[s])

        def wait_out(j, s):
            pltpu.make_async_copy(bufb.at[s], out_hbm.at[base + j],
                                  osems[s]).wait()

        for s in range(_NBUF):
            start_gather(s, s)

        def body(i, carry):
            for s in range(_NBUF):
                j = i * _NBUF + s
                wait_gather(j, s)
                repack(s)
                start_out(j, s)
            for s in range(_NBUF):
                j = i * _NBUF + s
                wait_out(j, s)
                start_gather(j + _NBUF, s)
            return carry

        n_grp = b_per_w // _NBUF
        lax.fori_loop(0, n_grp - 1, body, 0)
        for s in range(_NBUF):
            j = (n_grp - 1) * _NBUF + s
            wait_gather(j, s)
            repack(s)
            start_out(j, s)
        for s in range(_NBUF):
            j = (n_grp - 1) * _NBUF + s
            wait_out(j, s)

    return gk(tpad, cats)


def _project_transpose(g, Wt, bias_col, B, b0, x_prev):
    nb, L, E = g.shape
    O = Wt.shape[1]
    blk0 = b0 // _BB

    def body(g_ref, w_ref, b_ref, *rest):
        o_ref = rest[-1]
        w = w_ref[...]
        bb = b_ref[...]
        for l in range(L):
            x = g_ref[:, l, :]
            y = lax.dot_general(w, x, (((0,), (1,)), ((), ())),
                                preferred_element_type=jnp.float32)
            o_ref[l] = y + bb

    in_specs = [
        pl.BlockSpec((_BB, L, E), lambda i: (i, 0, 0)),
        pl.BlockSpec((E, O), lambda i: (0, 0)),
        pl.BlockSpec((O, 1), lambda i: (0, 0)),
    ]
    args = [g, Wt, bias_col]
    aliases = {}
    if x_prev is not None:
        in_specs.append(pl.BlockSpec(memory_space=pl.ANY))
        args.append(x_prev)
        aliases = {3: 0}

    return pl.pallas_call(
        body,
        grid=(nb // _BB,),
        in_specs=in_specs,
        out_specs=pl.BlockSpec((L, O, _BB), lambda i: (0, 0, blk0 + i)),
        out_shape=jax.ShapeDtypeStruct((L, O, B), jnp.float32),
        input_output_aliases=aliases,
    )(*args)


def kernel(categories, table, W, b):
    O, E = W.shape
    B, L = categories.shape
    tpad = jnp.pad(table, ((0, 0), (0, _LANE - E)))
    cats = categories.astype(jnp.int32)
    Wt = W.T
    bias_col = b.reshape(O, 1)

    nb = B // _NCHUNK
    x = None
    gs = [_gather_rows(tpad, cats, E, c * nb, nb) for c in range(_NCHUNK)]
    for c in range(_NCHUNK):
        x = _project_transpose(gs[c], Wt, bias_col, B, c * nb, x)
    return jnp.transpose(x, (2, 0, 1))


# speedup vs baseline: 7.5819x; 1.0227x over previous
import functools

import jax
import jax.numpy as jnp
from jax import lax
from jax.experimental import pallas as pl
from jax.experimental.pallas import tpu as pltpu
from jax.experimental.pallas import tpu_sc as plsc

_NBUF = 4
_LANE = 128
_BB = 512
_NCHUNK = 4


def _gather_rows(tpad, cats, E, b0, nb):
    B, L = cats.shape
    mesh = plsc.VectorSubcoreMesh(core_axis_name="c", subcore_axis_name="s")
    info = plsc.get_sparse_core_info()
    NC = info.num_cores
    NW = NC * info.num_subcores
    b_per_w = nb // NW

    @functools.partial(
        pl.kernel,
        out_type=jax.ShapeDtypeStruct((nb, L, E), jnp.float32),
        mesh=mesh,
        scratch_types=[
            pltpu.VMEM((b_per_w, L), jnp.int32),
            pltpu.VMEM((_NBUF, 56, _LANE), jnp.float32),
            pltpu.VMEM((_NBUF, L, E), jnp.float32),
            [pltpu.SemaphoreType.DMA] * _NBUF,
            [pltpu.SemaphoreType.DMA] * _NBUF,
        ],
        compiler_params=pltpu.CompilerParams(use_tc_tiling_on_sc=True),
    )
    def gk(tpad_hbm, cats_hbm, out_hbm, idx_v, buf, bufb, gsems, osems):
        wid = lax.axis_index("s") * NC + lax.axis_index("c")
        base = wid * b_per_w
        pltpu.sync_copy(cats_hbm.at[pl.ds(b0 + base, b_per_w)], idx_v)

        def start_gather(j, s):
            pltpu.async_copy(tpad_hbm.at[idx_v.at[j]], buf.at[s, :L], gsems[s])

        def wait_gather(j, s):
            pltpu.make_async_copy(tpad_hbm.at[idx_v.at[j]], buf.at[s, :L],
                                  gsems[s]).wait()

        def repack(s):
            def rbody(r, carry):
                for c in range(E // 16):
                    bufb[s, r, pl.ds(c * 16, 16)] = buf[s, r, pl.ds(c * 16, 16)]
                return carry

            lax.fori_loop(0, L, rbody, 0)

        def start_out(j, s):
            pltpu.async_copy(bufb.at[s], out_hbm.at[base + j], osems[s])

        def wait_out(j, s):
            pltpu.make_async_copy(bufb.at[s], out_hbm.at[base + j],
                                  osems[s]).wait()

        for s in range(_NBUF):
            start_gather(s, s)

        def body(i, carry):
            for s in range(_NBUF):
                j = i * _NBUF + s
                wait_gather(j, s)
                repack(s)
                start_out(j, s)
            for s in range(_NBUF):
                j = i * _NBUF + s
                wait_out(j, s)
                start_gather(j + _NBUF, s)
            return carry

        n_grp = b_per_w // _NBUF
        lax.fori_loop(0, n_grp - 1, body, 0)
        for s in range(_NBUF):
            j = (n_grp - 1) * _NBUF + s
            wait_gather(j, s)
            repack(s)
            start_out(j, s)
        for s in range(_NBUF):
            j = (n_grp - 1) * _NBUF + s
            wait_out(j, s)

    return gk(tpad, cats)


def _project_transpose(g, Wt, bias_col, B, b0, x_prev):
    nb, L, E = g.shape
    O = Wt.shape[1]
    blk0 = b0 // _BB

    def body(g_ref, w_ref, b_ref, *rest):
        o_ref = rest[-1]
        w = w_ref[...]
        bb = b_ref[...]
        for l in range(L):
            x = g_ref[:, l, :]
            y = lax.dot_general(w, x, (((0,), (1,)), ((), ())),
                                preferred_element_type=jnp.float32)
            o_ref[l] = y + bb

    in_specs = [
        pl.BlockSpec((_BB, L, E), lambda i: (i, 0, 0)),
        pl.BlockSpec((E, O), lambda i: (0, 0)),
        pl.BlockSpec((O, 1), lambda i: (0, 0)),
    ]
    args = [g, Wt, bias_col]
    aliases = {}
    if x_prev is not None:
        in_specs.append(pl.BlockSpec(memory_space=pl.ANY))
        args.append(x_prev)
        aliases = {3: 0}

    return pl.pallas_call(
        body,
        grid=(nb // _BB,),
        in_specs=in_specs,
        out_specs=pl.BlockSpec((L, O, _BB), lambda i: (0, 0, blk0 + i)),
        out_shape=jax.ShapeDtypeStruct((L, O, B), jnp.float32),
        input_output_aliases=aliases,
    )(*args)


def kernel(categories, table, W, b):
    O, E = W.shape
    B, L = categories.shape
    tpad = jnp.pad(table, ((0, 0), (0, _LANE - E)))
    cats = categories.astype(jnp.int32)
    Wt = W.T
    bias_col = b.reshape(O, 1)

    nb = B // _NCHUNK
    x = None
    gs = [_gather_rows(tpad, cats, E, c * nb, nb) for c in range(_NCHUNK)]
    for c in range(_NCHUNK):
        x = _project_transpose(gs[c], Wt, bias_col, B, c * nb, x)
    return jnp.transpose(x, (2, 0, 1))
